# baseline (speedup 1.0000x reference)
import functools

import jax
import jax.numpy as jnp
from jax import lax
from jax.experimental import pallas as pl
from jax.experimental.pallas import tpu as pltpu
from jax.experimental.pallas import tpu_sc as plsc

D = 128
BT = 4096
T = 50
NC, NS = 2, 16
NW = NC * NS
L = BT * T // NW
C = 128
K = 2
RROW = K * C
NR = L // C // K

_mesh = plsc.VectorSubcoreMesh(core_axis_name="c", subcore_axis_name="s")


@functools.partial(
    pl.kernel,
    out_type=jax.ShapeDtypeStruct((BT * T, D), jnp.float32),
    mesh=_mesh,
    scratch_types=[
        pltpu.VMEM((L,), jnp.int32),
        pltpu.VMEM((RROW, D), jnp.float32),
        pltpu.VMEM((RROW, D), jnp.float32),
        pltpu.VMEM((RROW, D), jnp.float32),
        pltpu.SemaphoreType.DMA,
        pltpu.SemaphoreType.DMA,
        pltpu.SemaphoreType.DMA,
        pltpu.SemaphoreType.DMA,
        pltpu.SemaphoreType.DMA,
        pltpu.SemaphoreType.DMA,
    ],
)
def _gather_kernel(idx_hbm, table_hbm, out_hbm,
                   idx_v, r0, r1, r2, g0, g1, g2, o0, o1, o2):
    bufs = (r0, r1, r2)
    sg = (g0, g1, g2)
    so = (o0, o1, o2)
    wid = lax.axis_index("s") * NC + lax.axis_index("c")
    base = wid * L
    pltpu.sync_copy(idx_hbm.at[wid], idx_v)

    def gather_start(r, s):
        for i in range(K):
            pltpu.make_async_copy(
                table_hbm.at[idx_v.at[pl.ds(r * RROW + i * C, C)]],
                bufs[s].at[pl.ds(i * C, C)], sg[s]).start()

    def gather_wait(s):
        pltpu.make_async_copy(
            table_hbm.at[idx_v.at[pl.ds(0, C)]], bufs[s], sg[s]).wait()

    def out_start(r, s):
        pltpu.make_async_copy(
            bufs[s], out_hbm.at[pl.ds(base + r * RROW, RROW)], so[s]).start()

    def out_wait(r, s):
        pltpu.make_async_copy(
            bufs[s], out_hbm.at[pl.ds(base + r * RROW, RROW)], so[s]).wait()

    gather_start(0, 0)
    gather_start(1, 1)
    gather_start(2, 2)
    gather_wait(0)
    out_start(0, 0)
    gather_wait(1)
    out_start(1, 1)

    def body(g, carry):
        for b in range(3):
            r = 2 + g * 3 + b
            out_wait(r - 2, b)
            gather_start(r + 1, b)
            gather_wait((2 + b) % 3)
            out_start(r, (2 + b) % 3)
        return carry

    lax.fori_loop(0, (NR - 4) // 3, body, 0)

    out_wait(NR - 4, 0)
    gather_start(NR - 1, 0)
    gather_wait(2)
    out_start(NR - 2, 2)

    out_wait(NR - 3, 1)
    gather_wait(0)
    out_start(NR - 1, 0)
    out_wait(NR - 2, 2)
    out_wait(NR - 1, 0)


def kernel(token_ids, weight):
    idx = token_ids.astype(jnp.int32).reshape(NW, L)
    return _gather_kernel(idx, weight).reshape(BT, T, D)


# speedup vs baseline: 1.0068x; 1.0068x over previous
import functools

import jax
import jax.numpy as jnp
from jax import lax
from jax.experimental import pallas as pl
from jax.experimental.pallas import tpu as pltpu
from jax.experimental.pallas import tpu_sc as plsc

D = 128
BT = 4096
T = 50
NC, NS = 2, 16
NW = NC * NS
L = BT * T // NW
C = 128
K = 3
RROW = K * C
NCHUNK = L // C
NR = NCHUNK // K
KR = NCHUNK - NR * K
RREM = KR * C

_mesh = plsc.VectorSubcoreMesh(core_axis_name="c", subcore_axis_name="s")


@functools.partial(
    pl.kernel,
    out_type=jax.ShapeDtypeStruct((BT * T, D), jnp.float32),
    mesh=_mesh,
    scratch_types=[
        pltpu.VMEM((L,), jnp.int32),
        pltpu.VMEM((RROW, D), jnp.float32),
        pltpu.VMEM((RROW, D), jnp.float32),
        pltpu.SemaphoreType.DMA,
        pltpu.SemaphoreType.DMA,
        pltpu.SemaphoreType.DMA,
        pltpu.SemaphoreType.DMA,
    ],
)
def _gather_kernel(idx_hbm, table_hbm, out_hbm,
                   idx_v, r0, r1, g0, g1, o0, o1):
    bufs = (r0, r1)
    sg = (g0, g1)
    so = (o0, o1)
    wid = lax.axis_index("s") * NC + lax.axis_index("c")
    base = wid * L
    pltpu.sync_copy(idx_hbm.at[wid], idx_v)

    def gather_start(r, s, k=K):
        for i in range(k):
            pltpu.make_async_copy(
                table_hbm.at[idx_v.at[pl.ds(r * RROW + i * C, C)]],
                bufs[s].at[pl.ds(i * C, C)], sg[s]).start()

    def gather_wait(s, rows=RROW):
        pltpu.make_async_copy(
            table_hbm.at[idx_v.at[pl.ds(0, C)]],
            bufs[s].at[pl.ds(0, rows)], sg[s]).wait()

    def out_start(r, s):
        pltpu.make_async_copy(
            bufs[s], out_hbm.at[pl.ds(base + r * RROW, RROW)], so[s]).start()

    def out_wait(r, s):
        pltpu.make_async_copy(
            bufs[s], out_hbm.at[pl.ds(base + r * RROW, RROW)], so[s]).wait()

    gather_start(0, 0)
    gather_start(1, 1)
    gather_wait(0)
    out_start(0, 0)

    def body(g, carry):
        for b in range(2):
            r = 1 + g * 2 + b
            out_wait(r - 1, b)
            gather_start(r + 1, b)
            gather_wait(1 - b)
            out_start(r, 1 - b)
        return carry

    lax.fori_loop(0, (NR - 2) // 2, body, 0)

    out_wait(NR - 2, 0)
    gather_start(NR, 0, k=KR)
    gather_wait(1)
    out_start(NR - 1, 1)
    out_wait(NR - 1, 1)

    gather_wait(0, rows=RREM)
    pltpu.make_async_copy(
        bufs[0].at[pl.ds(0, RREM)],
        out_hbm.at[pl.ds(base + NR * RROW, RREM)], so[0]).start()
    pltpu.make_async_copy(
        bufs[0].at[pl.ds(0, RREM)],
        out_hbm.at[pl.ds(base + NR * RROW, RREM)], so[0]).wait()


def kernel(token_ids, weight):
    idx = token_ids.astype(jnp.int32).reshape(NW, L)
    return _gather_kernel(idx, weight).reshape(BT, T, D)
